# baseline (speedup 1.0000x reference)
import functools

import jax
import jax.numpy as jnp
from jax import lax
from jax.experimental import pallas as pl
from jax.experimental.pallas import tpu as pltpu
from jax.experimental.pallas import tpu_sc as plsc

N = 10000
E = 320000
D = 128
NB = 64
NC = 2
NS = 16
HPC = D // NC
CH = 128
EPW = E // NS
NBUF = 4
CPW = -(-EPW // CH)
CPW = -(-CPW // NBUF) * NBUF
CPI = CPW + NBUF
A_ROWS = 10240
ZPS = A_ROWS // NS

_mesh = plsc.VectorSubcoreMesh(core_axis_name="c", subcore_axis_name="s")


@functools.partial(
    pl.kernel,
    out_type=jax.ShapeDtypeStruct((NC, A_ROWS, HPC), jnp.float32),
    mesh=_mesh,
    scratch_types=[
        pltpu.VMEM((CPI, CH), jnp.int32),
        pltpu.VMEM((CPI, CH), jnp.int32),
        [pltpu.VMEM((CH, HPC), jnp.float32)] * NBUF,
        pltpu.MemorySpace.VMEM_SHARED((A_ROWS, HPC), jnp.float32),
        [pltpu.SemaphoreType.DMA] * NBUF,
    ],
    compiler_params=pltpu.CompilerParams(use_tc_tiling_on_sc=False),
)
def _sc_edge_scatter(table, srcw, dstw, out, src_v, dst_v, rows, acc, sems):
    c = lax.axis_index("c")
    s = lax.axis_index("s")
    pltpu.sync_copy(srcw.at[c, s], src_v)
    pltpu.sync_copy(dstw.at[s], dst_v)

    zero = jnp.zeros((16,), jnp.float32)

    @pl.loop(0, CH)
    def _zero_row(i):
        for j in range(HPC // 16):
            rows[0][i, pl.ds(j * 16, 16)] = zero

    for t in range(ZPS // CH):
        pltpu.sync_copy(rows[0], acc.at[pl.ds(s * ZPS + t * CH, CH)])

    for b in range(NBUF):
        pltpu.async_copy(table.at[src_v.at[b]], rows[b], sems[b])
    plsc.subcore_barrier()

    @pl.loop(0, CPW // NBUF)
    def _grp(g):
        for b in range(NBUF):
            j = g * NBUF + b
            pltpu.make_async_copy(table.at[src_v.at[j]], rows[b], sems[b]).wait()
            pltpu.sync_copy(rows[b], acc.at[dst_v.at[j]], add=True)
            pltpu.async_copy(table.at[src_v.at[j + NBUF]], rows[b], sems[b])

    for b in range(NBUF):
        pltpu.make_async_copy(table.at[src_v.at[0]], rows[b], sems[b]).wait()

    plsc.subcore_barrier()
    for t in range(ZPS // CH):
        sl = pl.ds(s * ZPS + t * CH, CH)
        pltpu.sync_copy(acc.at[sl], out.at[c, sl])


def _split(hn):
    return hn[:, 0:HPC], hn[:, HPC:D]


def _merge(ap_ref):
    return jnp.concatenate([ap_ref[0, 0:N, :], ap_ref[1, 0:N, :]], axis=1)


def _tc_k0_body(x_ref, wn_ref, dp_ref, hn_ref, rc_ref):
    hn = jnp.dot(x_ref[...], wn_ref[...], preferred_element_type=jnp.float32)
    a, b = _split(hn)
    hn_ref[0] = a
    hn_ref[1] = b
    rc_ref[...] = 1.0 / jnp.maximum(_merge(dp_ref), 1.0)


_k0 = pl.pallas_call(
    _tc_k0_body,
    out_shape=(
        jax.ShapeDtypeStruct((NC, N, HPC), jnp.float32),
        jax.ShapeDtypeStruct((N, D), jnp.float32),
    ),
)


def _tc_combine_body(h_ref, ws_ref, bs_ref, bn_ref, wnn_ref, ap_ref, rc_ref,
                     hout_ref, hnout_ref):
    agg = _merge(ap_ref)
    z = (jnp.dot(h_ref[...], ws_ref[...], preferred_element_type=jnp.float32)
         + bs_ref[...] + agg * rc_ref[...] + bn_ref[...])
    hnew = jnp.maximum(z, 0.0)
    hout_ref[...] = hnew
    hn = jnp.dot(hnew, wnn_ref[...], preferred_element_type=jnp.float32)
    a, b = _split(hn)
    hnout_ref[0] = a
    hnout_ref[1] = b


_combine = pl.pallas_call(
    _tc_combine_body,
    out_shape=(
        jax.ShapeDtypeStruct((N, D), jnp.float32),
        jax.ShapeDtypeStruct((NC, N, HPC), jnp.float32),
    ),
)


def _tc_final_body(h_ref, ws_ref, bs_ref, bn_ref, ap_ref, rc_ref, bv_ref, out_ref):
    agg = _merge(ap_ref)
    z = (jnp.dot(h_ref[...], ws_ref[...], preferred_element_type=jnp.float32)
         + bs_ref[...] + agg * rc_ref[...] + bn_ref[...])
    h3 = jnp.maximum(z, 0.0)
    gid = lax.broadcasted_iota(jnp.int32, (1, NB), 1)
    oh = (bv_ref[...] == gid).astype(jnp.float32)
    ssum = lax.dot_general(oh, h3, (((0,), (0,)), ((), ())),
                           preferred_element_type=jnp.float32)
    cnt = lax.dot_general(oh, jnp.ones((N, 1), jnp.float32),
                          (((0,), (0,)), ((), ())),
                          preferred_element_type=jnp.float32)
    out_ref[...] = ssum / jnp.maximum(cnt, 1.0)


_final = pl.pallas_call(
    _tc_final_body,
    out_shape=jax.ShapeDtypeStruct((NB, D), jnp.float32),
)


def kernel(x, edge_index, batch_vec, Ws0, bs0, Wn0, bn0, Ws1, bs1, Wn1, bn1,
           Ws2, bs2, Wn2, bn2):
    src = edge_index[0].astype(jnp.int32)
    dst = edge_index[1].astype(jnp.int32)
    sp = (jnp.zeros((NS, CPI * CH), jnp.int32)
          .at[:, :EPW].set(src.reshape(NS, EPW)).reshape(NS, CPI, CH))
    srcw = jnp.stack([sp, sp + N])
    dstw = (jnp.full((NS, CPI * CH), N, jnp.int32)
            .at[:, :EPW].set(dst.reshape(NS, EPW)).reshape(NS, CPI, CH))
    bv = batch_vec.astype(jnp.int32).reshape(N, 1)

    degp = _sc_edge_scatter(jnp.ones((NC * N, HPC), jnp.float32), srcw, dstw)
    hn1, rc = _k0(x, Wn0, degp)
    agg1 = _sc_edge_scatter(hn1.reshape(NC * N, HPC), srcw, dstw)
    h1, hn2 = _combine(x, Ws0, bs0.reshape(1, D), bn0.reshape(1, D), Wn1, agg1, rc)
    agg2 = _sc_edge_scatter(hn2.reshape(NC * N, HPC), srcw, dstw)
    h2, hn3 = _combine(h1, Ws1, bs1.reshape(1, D), bn1.reshape(1, D), Wn2, agg2, rc)
    agg3 = _sc_edge_scatter(hn3.reshape(NC * N, HPC), srcw, dstw)
    out = _final(h2, Ws2, bs2.reshape(1, D), bn2.reshape(1, D), agg3, rc, bv)
    return out


# speedup vs baseline: 1.8743x; 1.8743x over previous
import functools

import jax
import jax.numpy as jnp
from jax import lax
from jax.experimental import pallas as pl
from jax.experimental.pallas import tpu as pltpu
from jax.experimental.pallas import tpu_sc as plsc

N = 10000
E = 320000
D = 128
NB = 64
NC = 2
NS = 16
NW = NC * NS
CH = 80
SLAB = 2
CPW = 126
CPI = CPW
NSL = CPW // SLAB
PAD_E = NW * CPI * CH
A_ROWS = 10112
ZPS = A_ROWS // NS
DW = 128

_mesh = plsc.VectorSubcoreMesh(core_axis_name="c", subcore_axis_name="s")


@functools.partial(
    pl.kernel,
    out_type=jax.ShapeDtypeStruct((NC, A_ROWS, D), jnp.float32),
    mesh=_mesh,
    scratch_types=[
        pltpu.VMEM((CPI * CH,), jnp.int32),
        pltpu.VMEM((CPI, CH), jnp.int32),
        pltpu.VMEM((SLAB * CH, D), jnp.float32),
        pltpu.MemorySpace.VMEM_SHARED((A_ROWS, D), jnp.float32),
        pltpu.SemaphoreType.DMA,
    ],
)
def _sc_edge_scatter(table, srcw, dstw, out, src_v, dst_v, rows_v, acc, sem):
    c = lax.axis_index("c")
    s = lax.axis_index("s")
    w = c * NS + s
    pltpu.sync_copy(srcw.at[w], src_v)
    pltpu.sync_copy(dstw.at[w], dst_v)

    zero = jnp.zeros((16,), jnp.float32)

    W = SLAB * CH

    @pl.loop(0, W)
    def _zero_row(i):
        for j in range(D // 16):
            rows_v[i, pl.ds(j * 16, 16)] = zero

    for t in range(ZPS // W):
        pltpu.sync_copy(rows_v.at[pl.ds(0, W)],
                        acc.at[pl.ds(s * ZPS + t * W, W)])
    if ZPS % W:
        pltpu.sync_copy(rows_v.at[pl.ds(0, ZPS % W)],
                        acc.at[pl.ds(s * ZPS + (ZPS // W) * W, ZPS % W)])
    plsc.subcore_barrier()

    @pl.loop(0, NSL)
    def _slab(g):
        pltpu.async_copy(table.at[src_v.at[pl.ds(g * W, W)]],
                         rows_v, sem).wait()
        for k in range(SLAB):
            pltpu.sync_copy(rows_v.at[pl.ds(k * CH, CH)],
                            acc.at[dst_v.at[g * SLAB + k]], add=True)

    plsc.subcore_barrier()
    for t in range(ZPS // W):
        sl = pl.ds(s * ZPS + t * W, W)
        pltpu.sync_copy(acc.at[sl], out.at[c, sl])
    if ZPS % W:
        sl = pl.ds(s * ZPS + (ZPS // W) * W, ZPS % W)
        pltpu.sync_copy(acc.at[sl], out.at[c, sl])


@functools.partial(
    pl.kernel,
    out_type=jax.ShapeDtypeStruct((NC, A_ROWS, DW), jnp.float32),
    mesh=_mesh,
    scratch_types=[
        pltpu.VMEM((CPI, CH), jnp.int32),
        pltpu.VMEM((CH, DW), jnp.float32),
        pltpu.MemorySpace.VMEM_SHARED((A_ROWS, DW), jnp.float32),
    ],
)
def _sc_degree(dstw, out, dst_v, ones_v, acc):
    c = lax.axis_index("c")
    s = lax.axis_index("s")
    w = c * NS + s
    pltpu.sync_copy(dstw.at[w], dst_v)

    one = jnp.ones((16,), jnp.float32)
    zero = jnp.zeros((16,), jnp.float32)

    @pl.loop(0, CH)
    def _zfill(i):
        for j in range(DW // 16):
            ones_v[i, pl.ds(j * 16, 16)] = zero

    for t in range(ZPS // CH):
        n = min(CH, ZPS - t * CH)
        pltpu.sync_copy(ones_v.at[pl.ds(0, n)],
                        acc.at[pl.ds(s * ZPS + t * CH, n)])
    if ZPS % CH:
        pltpu.sync_copy(ones_v.at[pl.ds(0, ZPS % CH)],
                        acc.at[pl.ds(s * ZPS + (ZPS // CH) * CH, ZPS % CH)])
    plsc.subcore_barrier()

    @pl.loop(0, CH)
    def _fill(i):
        for j in range(DW // 16):
            ones_v[i, pl.ds(j * 16, 16)] = one

    @pl.loop(0, CPW)
    def _chunk(j):
        pltpu.sync_copy(ones_v, acc.at[dst_v.at[j]], add=True)

    plsc.subcore_barrier()
    for t in range(ZPS // CH):
        n = min(CH, ZPS - t * CH)
        sl = pl.ds(s * ZPS + t * CH, n)
        pltpu.sync_copy(acc.at[sl], out.at[c, sl])
    if ZPS % CH:
        sl = pl.ds(s * ZPS + (ZPS // CH) * CH, ZPS % CH)
        pltpu.sync_copy(acc.at[sl], out.at[c, sl])


def _tc_k0_body(x_ref, wn_ref, dp_ref, hn_ref, rc_ref):
    hn_ref[...] = jnp.dot(x_ref[...], wn_ref[...], preferred_element_type=jnp.float32)
    rc_ref[...] = 1.0 / jnp.maximum(dp_ref[0, 0:N, :] + dp_ref[1, 0:N, :], 1.0)


_k0 = pl.pallas_call(
    _tc_k0_body,
    out_shape=(
        jax.ShapeDtypeStruct((N, D), jnp.float32),
        jax.ShapeDtypeStruct((N, D), jnp.float32),
    ),
)


def _tc_combine_body(h_ref, ws_ref, bs_ref, bn_ref, wnn_ref, ap_ref, rc_ref,
                     hout_ref, hnout_ref):
    agg = ap_ref[0, 0:N, :] + ap_ref[1, 0:N, :]
    z = (jnp.dot(h_ref[...], ws_ref[...], preferred_element_type=jnp.float32)
         + bs_ref[...] + agg * rc_ref[...] + bn_ref[...])
    hnew = jnp.maximum(z, 0.0)
    hout_ref[...] = hnew
    hnout_ref[...] = jnp.dot(hnew, wnn_ref[...], preferred_element_type=jnp.float32)


_combine = pl.pallas_call(
    _tc_combine_body,
    out_shape=(
        jax.ShapeDtypeStruct((N, D), jnp.float32),
        jax.ShapeDtypeStruct((N, D), jnp.float32),
    ),
)


def _tc_final_body(h_ref, ws_ref, bs_ref, bn_ref, ap_ref, rc_ref, bv_ref, out_ref):
    agg = ap_ref[0, 0:N, :] + ap_ref[1, 0:N, :]
    z = (jnp.dot(h_ref[...], ws_ref[...], preferred_element_type=jnp.float32)
         + bs_ref[...] + agg * rc_ref[...] + bn_ref[...])
    h3 = jnp.maximum(z, 0.0)
    gid = lax.broadcasted_iota(jnp.int32, (1, NB), 1)
    oh = (bv_ref[...] == gid).astype(jnp.float32)
    ssum = lax.dot_general(oh, h3, (((0,), (0,)), ((), ())),
                           preferred_element_type=jnp.float32)
    cnt = lax.dot_general(oh, jnp.ones((N, 1), jnp.float32),
                          (((0,), (0,)), ((), ())),
                          preferred_element_type=jnp.float32)
    out_ref[...] = ssum / jnp.maximum(cnt, 1.0)


_final = pl.pallas_call(
    _tc_final_body,
    out_shape=jax.ShapeDtypeStruct((NB, D), jnp.float32),
)


def kernel(x, edge_index, batch_vec, Ws0, bs0, Wn0, bn0, Ws1, bs1, Wn1, bn1,
           Ws2, bs2, Wn2, bn2):
    src = edge_index[0].astype(jnp.int32)
    dst = edge_index[1].astype(jnp.int32)
    srcw = (jnp.zeros((NW, CPI * CH), jnp.int32)
            .at[:, :E // NW].set(src.reshape(NW, E // NW)))
    dstw = (jnp.full((NW, CPI * CH), N, jnp.int32)
            .at[:, :E // NW].set(dst.reshape(NW, E // NW)).reshape(NW, CPI, CH))
    bv = batch_vec.astype(jnp.int32).reshape(N, 1)

    degp = _sc_degree(dstw)
    hn1, rc = _k0(x, Wn0, degp)
    agg1 = _sc_edge_scatter(hn1, srcw, dstw)
    h1, hn2 = _combine(x, Ws0, bs0.reshape(1, D), bn0.reshape(1, D), Wn1, agg1, rc)
    agg2 = _sc_edge_scatter(hn2, srcw, dstw)
    h2, hn3 = _combine(h1, Ws1, bs1.reshape(1, D), bn1.reshape(1, D), Wn2, agg2, rc)
    agg3 = _sc_edge_scatter(hn3, srcw, dstw)
    out = _final(h2, Ws2, bs2.reshape(1, D), bn2.reshape(1, D), agg3, rc, bv)
    return out
